# baseline (speedup 1.0000x reference)
import jax
import jax.numpy as jnp
from jax.experimental import pallas as pl
from jax.experimental.pallas import tpu as pltpu


def _roll8(t, k):
    return pltpu.roll(t, k % 8, axis=1)


def _rev_s(a):
    s, c = a.shape
    g = s // 8
    x4 = a.reshape(g, 8, c)
    io8 = jax.lax.broadcasted_iota(jnp.int32, (1, 8, 1), 1)
    r = _roll8(x4, 4)
    r = jnp.where((io8 % 4) < 2, _roll8(r, -2), _roll8(r, 2))
    r = jnp.where(io8 % 2 == 0, _roll8(r, -1), _roll8(r, 1))
    r = jnp.concatenate([r[t:t + 1] for t in range(g - 1, -1, -1)], axis=0)
    return r.reshape(s, c)


def _rev_s_blend_kernel(maskp_ref, x_ref, o_ref):
    bb = x_ref.shape[0]
    base = pl.program_id(0) * bb
    for i in range(bb):
        a = x_ref[i]
        keep = maskp_ref[base + i] != 0
        o_ref[i] = jnp.where(keep, a, _rev_s(a))


@jax.jit
def kernel(x, mask):
    B, C, H, W = x.shape
    S = H * W
    xs = jnp.transpose(x, (0, 2, 3, 1)).reshape(B, S, C)

    bb = 16 if B % 16 == 0 else B
    grid = (B // bb,)
    out = pl.pallas_call(
        _rev_s_blend_kernel,
        out_shape=jax.ShapeDtypeStruct((B, S, C), x.dtype),
        grid_spec=pltpu.PrefetchScalarGridSpec(
            num_scalar_prefetch=1,
            grid=grid,
            in_specs=[pl.BlockSpec((bb, S, C), lambda i, m: (i, 0, 0))],
            out_specs=pl.BlockSpec((bb, S, C), lambda i, m: (i, 0, 0)),
        ),
        compiler_params=pltpu.CompilerParams(
            dimension_semantics=("parallel",)),
    )(mask, xs)
    return jnp.transpose(out.reshape(B, H, W, C), (0, 3, 1, 2))


# speedup vs baseline: 1.0223x; 1.0223x over previous
import jax
import jax.numpy as jnp
from jax.experimental import pallas as pl
from jax.experimental.pallas import tpu as pltpu


def _roll8(t, k):
    return pltpu.roll(t, k % 8, axis=1)


def _rev_s(a):
    s, c = a.shape
    g = s // 8
    x4 = a.reshape(g, 8, c)
    io8 = jax.lax.broadcasted_iota(jnp.int32, (1, 8, 1), 1)
    r = _roll8(x4, 4)
    r = jnp.where((io8 % 4) < 2, _roll8(r, -2), _roll8(r, 2))
    r = jnp.where(io8 % 2 == 0, _roll8(r, -1), _roll8(r, 1))
    r = jnp.concatenate([r[t:t + 1] for t in range(g - 1, -1, -1)], axis=0)
    return r.reshape(s, c)


def _rev_s_blend_kernel(maskp_ref, x_ref, o_ref):
    bb = x_ref.shape[0]
    base = pl.program_id(0) * bb
    for i in range(bb):
        a = x_ref[i]
        keep = maskp_ref[base + i] != 0
        o_ref[i] = jnp.where(keep, a, _rev_s(a))


@jax.jit
def kernel(x, mask):
    B, C, H, W = x.shape
    S = H * W
    xs = jnp.transpose(x, (0, 2, 3, 1)).reshape(B, S, C)

    bb = 32 if B % 32 == 0 else B
    grid = (B // bb,)
    out = pl.pallas_call(
        _rev_s_blend_kernel,
        out_shape=jax.ShapeDtypeStruct((B, S, C), x.dtype),
        grid_spec=pltpu.PrefetchScalarGridSpec(
            num_scalar_prefetch=1,
            grid=grid,
            in_specs=[pl.BlockSpec((bb, S, C), lambda i, m: (i, 0, 0))],
            out_specs=pl.BlockSpec((bb, S, C), lambda i, m: (i, 0, 0)),
        ),
        compiler_params=pltpu.CompilerParams(
            dimension_semantics=("parallel",)),
    )(mask, xs)
    return jnp.transpose(out.reshape(B, H, W, C), (0, 3, 1, 2))
